# baseline (speedup 1.0000x reference)
import jax
import jax.numpy as jnp
from jax import lax
from jax.experimental import pallas as pl
from jax.experimental.pallas import tpu as pltpu
from jax.experimental.pallas import tpu_sc as plsc

_K = 26214
_ROWS = 128
_COLS = 32768
_CHUNKS = _COLS // 16
_ROWS_PER_SUBCORE = 4
_NHIST = 4
_HSTRIDE = 4096


def _ukey(b):
    sign = jnp.uint32(0x80000000)
    return jnp.where(b >= sign, ~b, b | sign)


def _sc_body(x_hbm, o_hbm, xbuf, hist, cand):
    c = lax.axis_index("c")
    s = lax.axis_index("s")
    wid = s * 2 + c
    lanes = lax.iota(jnp.int32, 16)
    ones = jnp.ones((16,), jnp.int32)

    for j in range(_ROWS_PER_SUBCORE):
        row = wid * _ROWS_PER_SUBCORE + j
        pltpu.sync_copy(x_hbm.at[row], xbuf)

        @plsc.parallel_loop(0, _NHIST * 256, unroll=8)
        def _zero(i):
            hist[pl.ds(i * 16, 16)] = jnp.zeros((16,), jnp.int32)

        @plsc.parallel_loop(0, _CHUNKS, unroll=4)
        def _hist(i):
            u = _ukey(xbuf[pl.ds(i * 16, 16)])
            d = (u >> jnp.uint32(24)).astype(jnp.int32)
            slot = d * jnp.int32(16) + lanes + (i & 3) * jnp.int32(_HSTRIDE)
            plsc.addupdate_scatter(hist, [slot], ones)

        def _scan(i, carry):
            cum, chosen, rnew = carry
            b = 255 - i
            base = b * 16
            hv = (
                hist[pl.ds(base, 16)]
                + hist[pl.ds(base + _HSTRIDE, 16)]
                + hist[pl.ds(base + 2 * _HSTRIDE, 16)]
                + hist[pl.ds(base + 3 * _HSTRIDE, 16)]
            )
            cum2 = cum + jnp.sum(hv)
            found = (cum < _K) & (cum2 >= _K)
            chosen = jnp.where(found, b, chosen)
            rnew = jnp.where(found, jnp.int32(_K) - cum, rnew)
            return (cum2, chosen, rnew)

        _, chosen, rank = plsc.parallel_loop(
            0, 256, unroll=4, carry=(jnp.int32(0), jnp.int32(0), jnp.int32(_K))
        )(_scan)
        prefix = chosen.astype(jnp.uint32) << jnp.uint32(24)

        def _cpt(i, off):
            u = _ukey(xbuf[pl.ds(i * 16, 16)])
            active = (u >> jnp.uint32(24)) == (prefix >> jnp.uint32(24))
            plsc.store_compressed(cand.at[pl.ds(off, 16)], u, mask=active)
            return off + plsc.all_reduce_population_count(active)[0]

        m = plsc.parallel_loop(0, _CHUNKS, unroll=4, carry=jnp.int32(0))(_cpt)
        cand[pl.ds(m, 16)] = jnp.zeros((16,), jnp.uint32)
        nch = (m + 15) >> 4

        def _bit(bi, t):
            bit = jnp.uint32(1) << (jnp.uint32(23) - bi.astype(jnp.uint32))
            candt = t | bit

            def _cnt(ci, acc):
                u = cand[pl.ds(ci * 16, 16)]
                return acc + jnp.where(u >= candt, jnp.int32(1), jnp.int32(0))

            acc = lax.fori_loop(0, nch, _cnt, jnp.zeros((16,), jnp.int32))
            return jnp.where(jnp.sum(acc) >= rank, candt, t)

        thresh = lax.fori_loop(0, 24, _bit, prefix)

        @plsc.parallel_loop(0, _CHUNKS, unroll=8)
        def _mask(i):
            sl = pl.ds(i * 16, 16)
            v = xbuf[sl]
            keep = _ukey(v) >= thresh
            xbuf[sl] = jnp.where(keep, v, jnp.uint32(0))

        pltpu.sync_copy(xbuf, o_hbm.at[row])


def kernel(x):
    f = pl.kernel(
        _sc_body,
        out_type=jax.ShapeDtypeStruct((_ROWS, _COLS), jnp.uint32),
        mesh=plsc.VectorSubcoreMesh(core_axis_name="c", subcore_axis_name="s"),
        compiler_params=pltpu.CompilerParams(needs_layout_passes=False),
        scratch_types=[
            pltpu.VMEM((_COLS,), jnp.uint32),
            pltpu.VMEM((_NHIST * _HSTRIDE,), jnp.int32),
            pltpu.VMEM((_COLS + 16,), jnp.uint32),
        ],
    )
    xu = jax.lax.bitcast_convert_type(x, jnp.uint32)
    return jax.lax.bitcast_convert_type(f(xu), jnp.float32)
